# baseline (speedup 1.0000x reference)
import numpy as np
import jax
from jax import lax
import jax.numpy as jnp
from jax.experimental import pallas as pl
from jax.experimental.pallas import tpu as pltpu
from jax.experimental.pallas import tpu_sc as plsc

CH = 32
CW = CH // 2
RSTRIDE = CW + 1
G = 32
NT = G * G
A = -0.75
NWORKERS = 32
CHUNK = 512
GRP = 16


def _cubic(t):
    t2 = t * t
    t3 = t2 * t
    w0 = A * (t3 - 2.0 * t2 + t)
    w1 = (A + 2.0) * t3 - (A + 3.0) * t2 + 1.0
    u = 1.0 - t
    u2 = u * u
    u3 = u2 * u
    w2 = (A + 2.0) * u3 - (A + 3.0) * u2 + 1.0
    w3 = A * (u3 - 2.0 * u2 + u)
    return (w0, w1, w2, w3)


def _axis_taps(v):
    s = v * (0.5 * (G - 1)) + (0.5 * (G - 1))
    i0 = s.astype(jnp.int32)
    t = s - i0.astype(jnp.float32)
    ws = _cubic(t)
    idx = tuple(jnp.clip(i0 + k, 0, G - 1) for k in (-1, 0, 1, 2))
    return idx, ws


def _sc_body(xs_hbm, ys_hbm, zs_hbm, tab_hbm, embt_hbm,
             xv, yv, zv, tab_v, out_v, sem_in, sem_out):
    npw = xs_hbm.shape[0] // NWORKERS
    nchunks = npw // CHUNK
    wid = lax.axis_index("s") * 2 + lax.axis_index("c")
    base = wid * npw
    pltpu.sync_copy(tab_hbm, tab_v)

    def start_in(ci, slot):
        off = base + ci * CHUNK
        pltpu.make_async_copy(
            xs_hbm.at[pl.ds(off, CHUNK)], xv.at[slot], sem_in).start()
        pltpu.make_async_copy(
            ys_hbm.at[pl.ds(off, CHUNK)], yv.at[slot], sem_in).start()
        pltpu.make_async_copy(
            zs_hbm.at[pl.ds(off, CHUNK)], zv.at[slot], sem_in).start()

    def drain_in(slot):
        pltpu.make_async_copy(
            xs_hbm.at[pl.ds(0, CHUNK)], xv.at[slot], sem_in).wait()
        pltpu.make_async_copy(
            ys_hbm.at[pl.ds(0, CHUNK)], yv.at[slot], sem_in).wait()
        pltpu.make_async_copy(
            zs_hbm.at[pl.ds(0, CHUNK)], zv.at[slot], sem_in).wait()

    def drain_out(slot):
        pltpu.make_async_copy(
            out_v.at[slot], embt_hbm.at[:, pl.ds(0, CHUNK)], sem_out).wait()

    start_in(0, 0)

    def chunk_body(ci, carry):
        slot = lax.rem(ci, 2)
        off = base + ci * CHUNK

        @pl.when(ci + 1 < nchunks)
        def _():
            start_in(ci + 1, 1 - slot)

        drain_in(slot)

        @pl.when(ci >= 2)
        def _():
            drain_out(slot)

        @plsc.parallel_loop(0, CHUNK // GRP)
        def group_body(g):
            xx = xv[slot, pl.ds(g * GRP, GRP)]
            yy = yv[slot, pl.ds(g * GRP, GRP)]
            zz = zv[slot, pl.ds(g * GRP, GRP)]
            xi, xw = _axis_taps(xx)
            yi, yw = _axis_taps(yy)
            zi, zw = _axis_taps(zz)
            accs = [jnp.zeros((2 * GRP,), jnp.bfloat16) for _ in range(CW)]
            planes = ((0, yi, yw, xi, xw),
                      (1, zi, zw, yi, yw),
                      (2, zi, zw, xi, xw))
            for p, ri, rw, ci_, cw in planes:
                poff = p * (NT * RSTRIDE)
                for j in range(4):
                    rowb = poff + ri[j] * (G * RSTRIDE)
                    for i in range(4):
                        bb = rowb + ci_[i] * RSTRIDE
                        w = rw[j] * cw[i]
                        wp = plsc.pack(w, w, format=plsc.PackFormat.INTERLEAVED)
                        for chw in range(CW):
                            gv = plsc.load_gather(tab_v, [bb + chw])
                            gb = plsc.bitcast(gv, jnp.bfloat16)
                            accs[chw] = accs[chw] + wp * gb
            for chw in range(CW):
                a, b = plsc.unpack(accs[chw],
                                   format=plsc.PackFormat.INTERLEAVED,
                                   preferred_element_type=jnp.float32)
                out_v[slot, 2 * chw, pl.ds(g * GRP, GRP)] = a
                out_v[slot, 2 * chw + 1, pl.ds(g * GRP, GRP)] = b

        pltpu.make_async_copy(
            out_v.at[slot], embt_hbm.at[:, pl.ds(off, CHUNK)], sem_out).start()
        return carry

    lax.fori_loop(0, nchunks, chunk_body, 0, unroll=False)
    drain_out(lax.rem(nchunks - 2, 2))
    drain_out(lax.rem(nchunks - 1, 2))


def _tail_body(embt_ref, bf_ref, o_ref):
    e = embt_ref[...]
    emb = e.T
    proj = jnp.dot(emb, bf_ref[...], preferred_element_type=jnp.float32)
    proj = proj * (2.0 * np.pi)
    o_ref[...] = jnp.concatenate([jnp.sin(proj), jnp.cos(proj)], axis=1)


def _axis_wmat(c, B):
    s = (c + 1.0) * (0.5 * (G - 1))
    s0 = jnp.floor(s)
    t = s - s0
    i0 = s0.astype(jnp.int32)
    ws = _cubic(t)
    cols = jax.lax.broadcasted_iota(jnp.int32, (B, G), 1)
    W = jnp.zeros((B, G), jnp.float32)
    for k in range(4):
        ik = jnp.clip(i0 + (k - 1), 0, G - 1)
        W = W + jnp.where(cols == ik[:, None], ws[k][:, None], 0.0)
    return W


def _tc_body(c_ref, tab_ref, bf_ref, o_ref):
    B = o_ref.shape[0]
    x = c_ref[0, :]
    y = c_ref[1, :]
    z = c_ref[2, :]
    Wx = _axis_wmat(x, B)
    Wy = _axis_wmat(y, B)
    Wz = _axis_wmat(z, B)
    Wpx = (Wy[:, :, None] * Wx[:, None, :]).reshape(B, NT)
    Wpy = (Wz[:, :, None] * Wy[:, None, :]).reshape(B, NT)
    Wpz = (Wz[:, :, None] * Wx[:, None, :]).reshape(B, NT)
    W3 = jnp.concatenate([Wpx, Wpy, Wpz], axis=1).astype(jnp.bfloat16)
    emb = jnp.dot(W3, tab_ref[...], preferred_element_type=jnp.float32)
    proj = jnp.dot(emb, bf_ref[...], preferred_element_type=jnp.float32)
    proj = proj * (2.0 * np.pi)
    o_ref[...] = jnp.concatenate([jnp.sin(proj), jnp.cos(proj)], axis=1)


N_TC = 65536


def kernel(coordinates, plane4_x, plane4_y, plane4_z, B_fourier,
           iteration=0, is_training=0):
    N = coordinates.shape[0]
    ct = coordinates.T
    n_tc = N_TC if N % NWORKERS == 0 and (N - N_TC) % (NWORKERS * CHUNK) == 0 else 0
    ct_tc, ct_sc = ct[:, :n_tc], ct[:, n_tc:]
    xs, ys, zs = ct_sc[0], ct_sc[1], ct_sc[2]
    N_sc = N - n_tc
    tab = jnp.concatenate(
        [jnp.transpose(p, (1, 2, 0)).reshape(-1)
         for p in (plane4_x, plane4_y, plane4_z)], axis=0)
    tabw = jax.lax.bitcast_convert_type(
        tab.astype(jnp.bfloat16).reshape(-1, 2), jnp.int32)
    tabw = jnp.pad(tabw.reshape(3 * NT, CW), ((0, 0), (0, RSTRIDE - CW))
                   ).reshape(-1)

    tabs_tc = jnp.concatenate(
        [jnp.transpose(p, (1, 2, 0)).reshape(NT, CH)
         for p in (plane4_x, plane4_y, plane4_z)], axis=0
    ).astype(jnp.bfloat16)
    B_TC = 1024
    out_tc = pl.pallas_call(
        _tc_body,
        grid=(n_tc // B_TC,),
        in_specs=[
            pl.BlockSpec((3, B_TC), lambda i: (0, i)),
            pl.BlockSpec((3 * NT, CH), lambda i: (0, 0)),
            pl.BlockSpec((CH, CH // 2), lambda i: (0, 0)),
        ],
        out_specs=pl.BlockSpec((B_TC, CH), lambda i: (i, 0)),
        out_shape=jax.ShapeDtypeStruct((n_tc, CH), jnp.float32),
    )(ct_tc, tabs_tc, B_fourier) if n_tc else None

    embt = pl.kernel(
        _sc_body,
        out_type=jax.ShapeDtypeStruct((CH, N_sc), jnp.float32),
        mesh=plsc.VectorSubcoreMesh(core_axis_name="c", subcore_axis_name="s"),
        compiler_params=pltpu.CompilerParams(needs_layout_passes=False),
        scratch_types=[
            pltpu.VMEM((2, CHUNK), jnp.float32),
            pltpu.VMEM((2, CHUNK), jnp.float32),
            pltpu.VMEM((2, CHUNK), jnp.float32),
            pltpu.VMEM((3 * NT * RSTRIDE,), jnp.int32),
            pltpu.VMEM((2, CH, CHUNK), jnp.float32),
            pltpu.SemaphoreType.DMA,
            pltpu.SemaphoreType.DMA,
        ],
    )(xs, ys, zs, tabw)

    B = 2048
    out_sc = pl.pallas_call(
        _tail_body,
        grid=(N_sc // B,),
        in_specs=[
            pl.BlockSpec((CH, B), lambda i: (0, i)),
            pl.BlockSpec((CH, CH // 2), lambda i: (0, 0)),
        ],
        out_specs=pl.BlockSpec((B, CH), lambda i: (i, 0)),
        out_shape=jax.ShapeDtypeStruct((N_sc, CH), jnp.float32),
    )(embt, B_fourier)
    if out_tc is None:
        return out_sc
    return jnp.concatenate([out_tc, out_sc], axis=0)


# speedup vs baseline: 1.4060x; 1.4060x over previous
import numpy as np
import jax
from jax import lax
import jax.numpy as jnp
from jax.experimental import pallas as pl
from jax.experimental.pallas import tpu as pltpu
from jax.experimental.pallas import tpu_sc as plsc

CH = 32
CW = CH // 2
RSTRIDE = CW + 1
G = 32
NT = G * G
A = -0.75
NWORKERS = 32
CHUNK = 512
GRP = 16


def _cubic(t):
    t2 = t * t
    t3 = t2 * t
    w0 = A * (t3 - 2.0 * t2 + t)
    w1 = (A + 2.0) * t3 - (A + 3.0) * t2 + 1.0
    u = 1.0 - t
    u2 = u * u
    u3 = u2 * u
    w2 = (A + 2.0) * u3 - (A + 3.0) * u2 + 1.0
    w3 = A * (u3 - 2.0 * u2 + u)
    return (w0, w1, w2, w3)


def _axis_taps(v):
    s = v * (0.5 * (G - 1)) + (0.5 * (G - 1))
    i0 = s.astype(jnp.int32)
    t = s - i0.astype(jnp.float32)
    ws = _cubic(t)
    idx = tuple(jnp.clip(i0 + k, 0, G - 1) for k in (-1, 0, 1, 2))
    return idx, ws


def _sc_body(xs_hbm, ys_hbm, zs_hbm, tab_hbm, embt_hbm,
             xv, yv, zv, tab_v, out_v, sem_in, sem_out):
    npw = xs_hbm.shape[0] // NWORKERS
    nchunks = npw // CHUNK
    wid = lax.axis_index("s") * 2 + lax.axis_index("c")
    base = wid * npw
    pltpu.sync_copy(tab_hbm, tab_v)

    def start_in(ci, slot):
        off = base + ci * CHUNK
        pltpu.make_async_copy(
            xs_hbm.at[pl.ds(off, CHUNK)], xv.at[slot], sem_in).start()
        pltpu.make_async_copy(
            ys_hbm.at[pl.ds(off, CHUNK)], yv.at[slot], sem_in).start()
        pltpu.make_async_copy(
            zs_hbm.at[pl.ds(off, CHUNK)], zv.at[slot], sem_in).start()

    def drain_in(slot):
        pltpu.make_async_copy(
            xs_hbm.at[pl.ds(0, CHUNK)], xv.at[slot], sem_in).wait()
        pltpu.make_async_copy(
            ys_hbm.at[pl.ds(0, CHUNK)], yv.at[slot], sem_in).wait()
        pltpu.make_async_copy(
            zs_hbm.at[pl.ds(0, CHUNK)], zv.at[slot], sem_in).wait()

    def drain_out(slot):
        pltpu.make_async_copy(
            out_v.at[slot], embt_hbm.at[:, pl.ds(0, CHUNK)], sem_out).wait()

    start_in(0, 0)

    def chunk_body(ci, carry):
        slot = lax.rem(ci, 2)
        off = base + ci * CHUNK

        @pl.when(ci + 1 < nchunks)
        def _():
            start_in(ci + 1, 1 - slot)

        drain_in(slot)

        @pl.when(ci >= 2)
        def _():
            drain_out(slot)

        @plsc.parallel_loop(0, CHUNK // GRP)
        def group_body(g):
            xx = xv[slot, pl.ds(g * GRP, GRP)]
            yy = yv[slot, pl.ds(g * GRP, GRP)]
            zz = zv[slot, pl.ds(g * GRP, GRP)]
            xi, xw = _axis_taps(xx)
            yi, yw = _axis_taps(yy)
            zi, zw = _axis_taps(zz)
            accs = [jnp.zeros((2 * GRP,), jnp.bfloat16) for _ in range(CW)]
            planes = ((0, yi, yw, xi, xw),
                      (1, zi, zw, yi, yw),
                      (2, zi, zw, xi, xw))
            for p, ri, rw, ci_, cw in planes:
                poff = p * (NT * RSTRIDE)
                for j in range(4):
                    rowb = poff + ri[j] * (G * RSTRIDE)
                    for i in range(4):
                        bb = rowb + ci_[i] * RSTRIDE
                        w = rw[j] * cw[i]
                        wp = plsc.pack(w, w, format=plsc.PackFormat.INTERLEAVED)
                        for chw in range(CW):
                            gv = plsc.load_gather(tab_v, [bb + chw])
                            gb = plsc.bitcast(gv, jnp.bfloat16)
                            accs[chw] = accs[chw] + wp * gb
            for chw in range(CW):
                out_v[slot, chw, pl.ds(g * GRP, GRP)] = plsc.bitcast(
                    accs[chw], jnp.int32)

        pltpu.make_async_copy(
            out_v.at[slot], embt_hbm.at[:, pl.ds(off, CHUNK)], sem_out).start()
        return carry

    lax.fori_loop(0, nchunks, chunk_body, 0, unroll=False)
    drain_out(lax.rem(nchunks - 2, 2))
    drain_out(lax.rem(nchunks - 1, 2))


def _tail_body(embt_ref, bfe_ref, bfo_ref, o_ref):
    e = embt_ref[...]
    ee = jax.lax.bitcast_convert_type(e << 16, jnp.float32).T
    eo = jax.lax.bitcast_convert_type(
        e & jnp.int32(-65536), jnp.float32).T
    proj = (jnp.dot(ee, bfe_ref[...], preferred_element_type=jnp.float32)
            + jnp.dot(eo, bfo_ref[...], preferred_element_type=jnp.float32))
    proj = proj * (2.0 * np.pi)
    o_ref[...] = jnp.concatenate([jnp.sin(proj), jnp.cos(proj)], axis=1)


def _axis_wmat(c, B):
    s = (c + 1.0) * (0.5 * (G - 1))
    s0 = jnp.floor(s)
    t = s - s0
    i0 = s0.astype(jnp.int32)
    ws = _cubic(t)
    cols = jax.lax.broadcasted_iota(jnp.int32, (B, G), 1)
    W = jnp.zeros((B, G), jnp.float32)
    for k in range(4):
        ik = jnp.clip(i0 + (k - 1), 0, G - 1)
        W = W + jnp.where(cols == ik[:, None], ws[k][:, None], 0.0)
    return W


def _tc_body(c_ref, tab_ref, bf_ref, o_ref):
    B = o_ref.shape[0]
    x = c_ref[0, :]
    y = c_ref[1, :]
    z = c_ref[2, :]
    Wx = _axis_wmat(x, B)
    Wy = _axis_wmat(y, B)
    Wz = _axis_wmat(z, B)
    Wpx = (Wy[:, :, None] * Wx[:, None, :]).reshape(B, NT)
    Wpy = (Wz[:, :, None] * Wy[:, None, :]).reshape(B, NT)
    Wpz = (Wz[:, :, None] * Wx[:, None, :]).reshape(B, NT)
    W3 = jnp.concatenate([Wpx, Wpy, Wpz], axis=1).astype(jnp.bfloat16)
    emb = jnp.dot(W3, tab_ref[...], preferred_element_type=jnp.float32)
    proj = jnp.dot(emb, bf_ref[...], preferred_element_type=jnp.float32)
    proj = proj * (2.0 * np.pi)
    o_ref[...] = jnp.concatenate([jnp.sin(proj), jnp.cos(proj)], axis=1)


N_TC = 65536


def kernel(coordinates, plane4_x, plane4_y, plane4_z, B_fourier,
           iteration=0, is_training=0):
    N = coordinates.shape[0]
    ct = coordinates.T
    n_tc = 0
    ct_tc, ct_sc = ct[:, :n_tc], ct[:, n_tc:]
    xs, ys, zs = ct_sc[0], ct_sc[1], ct_sc[2]
    N_sc = N - n_tc
    tab = jnp.concatenate(
        [jnp.transpose(p, (1, 2, 0)).reshape(-1)
         for p in (plane4_x, plane4_y, plane4_z)], axis=0)
    tabw = jax.lax.bitcast_convert_type(
        tab.astype(jnp.bfloat16).reshape(-1, 2), jnp.int32)
    tabw = jnp.pad(tabw.reshape(3 * NT, CW), ((0, 0), (0, RSTRIDE - CW))
                   ).reshape(-1)

    tabs_tc = jnp.concatenate(
        [jnp.transpose(p, (1, 2, 0)).reshape(NT, CH)
         for p in (plane4_x, plane4_y, plane4_z)], axis=0
    ).astype(jnp.bfloat16)
    B_TC = 1024
    out_tc = pl.pallas_call(
        _tc_body,
        grid=(n_tc // B_TC,),
        in_specs=[
            pl.BlockSpec((3, B_TC), lambda i: (0, i)),
            pl.BlockSpec((3 * NT, CH), lambda i: (0, 0)),
            pl.BlockSpec((CH, CH // 2), lambda i: (0, 0)),
        ],
        out_specs=pl.BlockSpec((B_TC, CH), lambda i: (i, 0)),
        out_shape=jax.ShapeDtypeStruct((n_tc, CH), jnp.float32),
    )(ct_tc, tabs_tc, B_fourier) if n_tc else None

    embt = pl.kernel(
        _sc_body,
        out_type=jax.ShapeDtypeStruct((CW, N_sc), jnp.int32),
        mesh=plsc.VectorSubcoreMesh(core_axis_name="c", subcore_axis_name="s"),
        compiler_params=pltpu.CompilerParams(needs_layout_passes=False),
        scratch_types=[
            pltpu.VMEM((2, CHUNK), jnp.float32),
            pltpu.VMEM((2, CHUNK), jnp.float32),
            pltpu.VMEM((2, CHUNK), jnp.float32),
            pltpu.VMEM((3 * NT * RSTRIDE,), jnp.int32),
            pltpu.VMEM((2, CW, CHUNK), jnp.int32),
            pltpu.SemaphoreType.DMA,
            pltpu.SemaphoreType.DMA,
        ],
    )(xs, ys, zs, tabw)

    B = 2048
    bfe = B_fourier[0::2, :]
    bfo = B_fourier[1::2, :]
    out_sc = pl.pallas_call(
        _tail_body,
        grid=(N_sc // B,),
        in_specs=[
            pl.BlockSpec((CW, B), lambda i: (0, i)),
            pl.BlockSpec((CW, CH // 2), lambda i: (0, 0)),
            pl.BlockSpec((CW, CH // 2), lambda i: (0, 0)),
        ],
        out_specs=pl.BlockSpec((B, CH), lambda i: (i, 0)),
        out_shape=jax.ShapeDtypeStruct((N_sc, CH), jnp.float32),
    )(embt, bfe, bfo)
    if out_tc is None:
        return out_sc
    return jnp.concatenate([out_tc, out_sc], axis=0)


# speedup vs baseline: 1.4063x; 1.0002x over previous
import numpy as np
import jax
from jax import lax
import jax.numpy as jnp
from jax.experimental import pallas as pl
from jax.experimental.pallas import tpu as pltpu
from jax.experimental.pallas import tpu_sc as plsc

CH = 32
CW = CH // 2
RSTRIDE = CW + 1
G = 32
NT = G * G
A = -0.75
NWORKERS = 32
CHUNK = 1024
GRP = 16


def _cubic(t):
    t2 = t * t
    t3 = t2 * t
    w0 = A * (t3 - 2.0 * t2 + t)
    w1 = (A + 2.0) * t3 - (A + 3.0) * t2 + 1.0
    u = 1.0 - t
    u2 = u * u
    u3 = u2 * u
    w2 = (A + 2.0) * u3 - (A + 3.0) * u2 + 1.0
    w3 = A * (u3 - 2.0 * u2 + u)
    return (w0, w1, w2, w3)


def _axis_taps(v):
    s = v * (0.5 * (G - 1)) + (0.5 * (G - 1))
    i0 = s.astype(jnp.int32)
    t = s - i0.astype(jnp.float32)
    ws = _cubic(t)
    idx = tuple(jnp.clip(i0 + k, 0, G - 1) for k in (-1, 0, 1, 2))
    return idx, ws


def _sc_body(xs_hbm, ys_hbm, zs_hbm, tab_hbm, embt_hbm,
             xv, yv, zv, tab_v, out_v, sem_in, sem_out):
    npw = xs_hbm.shape[0] // NWORKERS
    nchunks = npw // CHUNK
    wid = lax.axis_index("s") * 2 + lax.axis_index("c")
    base = wid * npw
    pltpu.sync_copy(tab_hbm, tab_v)

    def start_in(ci, slot):
        off = base + ci * CHUNK
        pltpu.make_async_copy(
            xs_hbm.at[pl.ds(off, CHUNK)], xv.at[slot], sem_in).start()
        pltpu.make_async_copy(
            ys_hbm.at[pl.ds(off, CHUNK)], yv.at[slot], sem_in).start()
        pltpu.make_async_copy(
            zs_hbm.at[pl.ds(off, CHUNK)], zv.at[slot], sem_in).start()

    def drain_in(slot):
        pltpu.make_async_copy(
            xs_hbm.at[pl.ds(0, CHUNK)], xv.at[slot], sem_in).wait()
        pltpu.make_async_copy(
            ys_hbm.at[pl.ds(0, CHUNK)], yv.at[slot], sem_in).wait()
        pltpu.make_async_copy(
            zs_hbm.at[pl.ds(0, CHUNK)], zv.at[slot], sem_in).wait()

    def drain_out(slot):
        pltpu.make_async_copy(
            out_v.at[slot], embt_hbm.at[:, pl.ds(0, CHUNK)], sem_out).wait()

    start_in(0, 0)

    def chunk_body(ci, carry):
        slot = lax.rem(ci, 2)
        off = base + ci * CHUNK

        @pl.when(ci + 1 < nchunks)
        def _():
            start_in(ci + 1, 1 - slot)

        drain_in(slot)

        @pl.when(ci >= 2)
        def _():
            drain_out(slot)

        @plsc.parallel_loop(0, CHUNK // GRP)
        def group_body(g):
            xx = xv[slot, pl.ds(g * GRP, GRP)]
            yy = yv[slot, pl.ds(g * GRP, GRP)]
            zz = zv[slot, pl.ds(g * GRP, GRP)]
            xi, xw = _axis_taps(xx)
            yi, yw = _axis_taps(yy)
            zi, zw = _axis_taps(zz)
            accs = [jnp.zeros((2 * GRP,), jnp.bfloat16) for _ in range(CW)]
            planes = ((0, yi, yw, xi, xw),
                      (1, zi, zw, yi, yw),
                      (2, zi, zw, xi, xw))
            for p, ri, rw, ci_, cw in planes:
                poff = p * (NT * RSTRIDE)
                for j in range(4):
                    rowb = poff + ri[j] * (G * RSTRIDE)
                    for i in range(4):
                        bb = rowb + ci_[i] * RSTRIDE
                        w = rw[j] * cw[i]
                        wp = plsc.pack(w, w, format=plsc.PackFormat.INTERLEAVED)
                        for chw in range(CW):
                            gv = plsc.load_gather(tab_v, [bb + chw])
                            gb = plsc.bitcast(gv, jnp.bfloat16)
                            accs[chw] = accs[chw] + wp * gb
            for chw in range(CW):
                out_v[slot, chw, pl.ds(g * GRP, GRP)] = plsc.bitcast(
                    accs[chw], jnp.int32)

        pltpu.make_async_copy(
            out_v.at[slot], embt_hbm.at[:, pl.ds(off, CHUNK)], sem_out).start()
        return carry

    lax.fori_loop(0, nchunks, chunk_body, 0, unroll=False)
    drain_out(lax.rem(nchunks - 2, 2))
    drain_out(lax.rem(nchunks - 1, 2))


def _tail_body(embt_ref, bfe_ref, bfo_ref, o_ref):
    e = embt_ref[...]
    ee = jax.lax.bitcast_convert_type(e << 16, jnp.float32).T
    eo = jax.lax.bitcast_convert_type(
        e & jnp.int32(-65536), jnp.float32).T
    proj = (jnp.dot(ee, bfe_ref[...], preferred_element_type=jnp.float32)
            + jnp.dot(eo, bfo_ref[...], preferred_element_type=jnp.float32))
    proj = proj * (2.0 * np.pi)
    o_ref[...] = jnp.concatenate([jnp.sin(proj), jnp.cos(proj)], axis=1)


def _axis_wmat(c, B):
    s = (c + 1.0) * (0.5 * (G - 1))
    s0 = jnp.floor(s)
    t = s - s0
    i0 = s0.astype(jnp.int32)
    ws = _cubic(t)
    cols = jax.lax.broadcasted_iota(jnp.int32, (B, G), 1)
    W = jnp.zeros((B, G), jnp.float32)
    for k in range(4):
        ik = jnp.clip(i0 + (k - 1), 0, G - 1)
        W = W + jnp.where(cols == ik[:, None], ws[k][:, None], 0.0)
    return W


def _tc_body(c_ref, tab_ref, bf_ref, o_ref):
    B = o_ref.shape[0]
    x = c_ref[0, :]
    y = c_ref[1, :]
    z = c_ref[2, :]
    Wx = _axis_wmat(x, B)
    Wy = _axis_wmat(y, B)
    Wz = _axis_wmat(z, B)
    Wpx = (Wy[:, :, None] * Wx[:, None, :]).reshape(B, NT)
    Wpy = (Wz[:, :, None] * Wy[:, None, :]).reshape(B, NT)
    Wpz = (Wz[:, :, None] * Wx[:, None, :]).reshape(B, NT)
    W3 = jnp.concatenate([Wpx, Wpy, Wpz], axis=1).astype(jnp.bfloat16)
    emb = jnp.dot(W3, tab_ref[...], preferred_element_type=jnp.float32)
    proj = jnp.dot(emb, bf_ref[...], preferred_element_type=jnp.float32)
    proj = proj * (2.0 * np.pi)
    o_ref[...] = jnp.concatenate([jnp.sin(proj), jnp.cos(proj)], axis=1)


N_TC = 65536


def kernel(coordinates, plane4_x, plane4_y, plane4_z, B_fourier,
           iteration=0, is_training=0):
    N = coordinates.shape[0]
    ct = coordinates.T
    n_tc = 0
    ct_tc, ct_sc = ct[:, :n_tc], ct[:, n_tc:]
    xs, ys, zs = ct_sc[0], ct_sc[1], ct_sc[2]
    N_sc = N - n_tc
    tab = jnp.concatenate(
        [jnp.transpose(p, (1, 2, 0)).reshape(-1)
         for p in (plane4_x, plane4_y, plane4_z)], axis=0)
    tabw = jax.lax.bitcast_convert_type(
        tab.astype(jnp.bfloat16).reshape(-1, 2), jnp.int32)
    tabw = jnp.pad(tabw.reshape(3 * NT, CW), ((0, 0), (0, RSTRIDE - CW))
                   ).reshape(-1)

    tabs_tc = jnp.concatenate(
        [jnp.transpose(p, (1, 2, 0)).reshape(NT, CH)
         for p in (plane4_x, plane4_y, plane4_z)], axis=0
    ).astype(jnp.bfloat16)
    B_TC = 1024
    out_tc = pl.pallas_call(
        _tc_body,
        grid=(n_tc // B_TC,),
        in_specs=[
            pl.BlockSpec((3, B_TC), lambda i: (0, i)),
            pl.BlockSpec((3 * NT, CH), lambda i: (0, 0)),
            pl.BlockSpec((CH, CH // 2), lambda i: (0, 0)),
        ],
        out_specs=pl.BlockSpec((B_TC, CH), lambda i: (i, 0)),
        out_shape=jax.ShapeDtypeStruct((n_tc, CH), jnp.float32),
    )(ct_tc, tabs_tc, B_fourier) if n_tc else None

    embt = pl.kernel(
        _sc_body,
        out_type=jax.ShapeDtypeStruct((CW, N_sc), jnp.int32),
        mesh=plsc.VectorSubcoreMesh(core_axis_name="c", subcore_axis_name="s"),
        compiler_params=pltpu.CompilerParams(needs_layout_passes=False),
        scratch_types=[
            pltpu.VMEM((2, CHUNK), jnp.float32),
            pltpu.VMEM((2, CHUNK), jnp.float32),
            pltpu.VMEM((2, CHUNK), jnp.float32),
            pltpu.VMEM((3 * NT * RSTRIDE,), jnp.int32),
            pltpu.VMEM((2, CW, CHUNK), jnp.int32),
            pltpu.SemaphoreType.DMA,
            pltpu.SemaphoreType.DMA,
        ],
    )(xs, ys, zs, tabw)

    B = 2048
    bfe = B_fourier[0::2, :]
    bfo = B_fourier[1::2, :]
    out_sc = pl.pallas_call(
        _tail_body,
        grid=(N_sc // B,),
        in_specs=[
            pl.BlockSpec((CW, B), lambda i: (0, i)),
            pl.BlockSpec((CW, CH // 2), lambda i: (0, 0)),
            pl.BlockSpec((CW, CH // 2), lambda i: (0, 0)),
        ],
        out_specs=pl.BlockSpec((B, CH), lambda i: (i, 0)),
        out_shape=jax.ShapeDtypeStruct((N_sc, CH), jnp.float32),
    )(embt, bfe, bfo)
    if out_tc is None:
        return out_sc
    return jnp.concatenate([out_tc, out_sc], axis=0)


# speedup vs baseline: 1.4154x; 1.0065x over previous
import numpy as np
import jax
from jax import lax
import jax.numpy as jnp
from jax.experimental import pallas as pl
from jax.experimental.pallas import tpu as pltpu
from jax.experimental.pallas import tpu_sc as plsc

CH = 32
CW = CH // 2
RSTRIDE = CW + 1
G = 32
NT = G * G
A = -0.75
NWORKERS = 32
CHUNK = 512
GRP = 16


def _cubic(t):
    t2 = t * t
    t3 = t2 * t
    w0 = A * (t3 - 2.0 * t2 + t)
    w1 = (A + 2.0) * t3 - (A + 3.0) * t2 + 1.0
    u = 1.0 - t
    u2 = u * u
    u3 = u2 * u
    w2 = (A + 2.0) * u3 - (A + 3.0) * u2 + 1.0
    w3 = A * (u3 - 2.0 * u2 + u)
    return (w0, w1, w2, w3)


def _axis_taps(v):
    s = v * (0.5 * (G - 1)) + (0.5 * (G - 1))
    i0 = s.astype(jnp.int32)
    t = s - i0.astype(jnp.float32)
    ws = _cubic(t)
    idx = tuple(jnp.clip(i0 + k, 0, G - 1) for k in (-1, 0, 1, 2))
    return idx, ws


def _sc_body(xs_hbm, ys_hbm, zs_hbm, tab_hbm, embt_hbm,
             xv, yv, zv, tab_v, out_v, sem_in, sem_out):
    npw = xs_hbm.shape[0] // NWORKERS
    nchunks = npw // CHUNK
    wid = lax.axis_index("s") * 2 + lax.axis_index("c")
    base = wid * npw
    pltpu.sync_copy(tab_hbm, tab_v)

    def start_in(ci, slot):
        off = base + ci * CHUNK
        pltpu.make_async_copy(
            xs_hbm.at[pl.ds(off, CHUNK)], xv.at[slot], sem_in).start()
        pltpu.make_async_copy(
            ys_hbm.at[pl.ds(off, CHUNK)], yv.at[slot], sem_in).start()
        pltpu.make_async_copy(
            zs_hbm.at[pl.ds(off, CHUNK)], zv.at[slot], sem_in).start()

    def drain_in(slot):
        pltpu.make_async_copy(
            xs_hbm.at[pl.ds(0, CHUNK)], xv.at[slot], sem_in).wait()
        pltpu.make_async_copy(
            ys_hbm.at[pl.ds(0, CHUNK)], yv.at[slot], sem_in).wait()
        pltpu.make_async_copy(
            zs_hbm.at[pl.ds(0, CHUNK)], zv.at[slot], sem_in).wait()

    def drain_out(slot):
        pltpu.make_async_copy(
            out_v.at[slot], embt_hbm.at[:, pl.ds(0, CHUNK)], sem_out).wait()

    start_in(0, 0)

    def chunk_body(ci, carry):
        slot = lax.rem(ci, 2)
        off = base + ci * CHUNK

        @pl.when(ci + 1 < nchunks)
        def _():
            start_in(ci + 1, 1 - slot)

        drain_in(slot)

        @pl.when(ci >= 2)
        def _():
            drain_out(slot)

        @plsc.parallel_loop(0, CHUNK // GRP)
        def group_body(g):
            xx = xv[slot, pl.ds(g * GRP, GRP)]
            yy = yv[slot, pl.ds(g * GRP, GRP)]
            zz = zv[slot, pl.ds(g * GRP, GRP)]
            xi, xw = _axis_taps(xx)
            yi, yw = _axis_taps(yy)
            zi, zw = _axis_taps(zz)
            accs = [jnp.zeros((2 * GRP,), jnp.bfloat16) for _ in range(CW)]
            planes = ((0, yi, yw, xi, xw),
                      (1, zi, zw, yi, yw),
                      (2, zi, zw, xi, xw))
            for p, ri, rw, ci_, cw in planes:
                poff = p * (NT * RSTRIDE)
                for j in range(4):
                    rowb = poff + ri[j] * (G * RSTRIDE)
                    for i in range(4):
                        bb = rowb + ci_[i] * RSTRIDE
                        w = rw[j] * cw[i]
                        wp = plsc.pack(w, w, format=plsc.PackFormat.INTERLEAVED)
                        for chw in range(CW):
                            gv = plsc.load_gather(tab_v, [bb + chw])
                            gb = plsc.bitcast(gv, jnp.bfloat16)
                            accs[chw] = accs[chw] + wp * gb
            for chw in range(CW):
                a, b = plsc.unpack(accs[chw],
                                   format=plsc.PackFormat.INTERLEAVED,
                                   preferred_element_type=jnp.float32)
                out_v[slot, 2 * chw, pl.ds(g * GRP, GRP)] = a
                out_v[slot, 2 * chw + 1, pl.ds(g * GRP, GRP)] = b

        pltpu.make_async_copy(
            out_v.at[slot], embt_hbm.at[:, pl.ds(off, CHUNK)], sem_out).start()
        return carry

    lax.fori_loop(0, nchunks, chunk_body, 0, unroll=False)
    drain_out(lax.rem(nchunks - 2, 2))
    drain_out(lax.rem(nchunks - 1, 2))


def _tail_body(embt_ref, bf_ref, o_ref):
    e = embt_ref[...]
    emb = e.T
    proj = jnp.dot(emb, bf_ref[...], preferred_element_type=jnp.float32)
    proj = proj * (2.0 * np.pi)
    o_ref[...] = jnp.concatenate([jnp.sin(proj), jnp.cos(proj)], axis=1)


def kernel(coordinates, plane4_x, plane4_y, plane4_z, B_fourier,
           iteration=0, is_training=0):
    N = coordinates.shape[0]
    ct = coordinates.T
    xs, ys, zs = ct[0], ct[1], ct[2]
    tab = jnp.concatenate(
        [jnp.transpose(p, (1, 2, 0)).reshape(-1)
         for p in (plane4_x, plane4_y, plane4_z)], axis=0)
    tabw = jax.lax.bitcast_convert_type(
        tab.astype(jnp.bfloat16).reshape(-1, 2), jnp.int32)
    tabw = jnp.pad(tabw.reshape(3 * NT, CW), ((0, 0), (0, RSTRIDE - CW))
                   ).reshape(-1)

    embt = pl.kernel(
        _sc_body,
        out_type=jax.ShapeDtypeStruct((CH, N), jnp.float32),
        mesh=plsc.VectorSubcoreMesh(core_axis_name="c", subcore_axis_name="s"),
        compiler_params=pltpu.CompilerParams(needs_layout_passes=False),
        scratch_types=[
            pltpu.VMEM((2, CHUNK), jnp.float32),
            pltpu.VMEM((2, CHUNK), jnp.float32),
            pltpu.VMEM((2, CHUNK), jnp.float32),
            pltpu.VMEM((3 * NT * RSTRIDE,), jnp.int32),
            pltpu.VMEM((2, CH, CHUNK), jnp.float32),
            pltpu.SemaphoreType.DMA,
            pltpu.SemaphoreType.DMA,
        ],
    )(xs, ys, zs, tabw)

    B = 2048
    out = pl.pallas_call(
        _tail_body,
        grid=(N // B,),
        in_specs=[
            pl.BlockSpec((CH, B), lambda i: (0, i)),
            pl.BlockSpec((CH, CH // 2), lambda i: (0, 0)),
        ],
        out_specs=pl.BlockSpec((B, CH), lambda i: (i, 0)),
        out_shape=jax.ShapeDtypeStruct((N, CH), jnp.float32),
    )(embt, B_fourier)
    return out
